# Initial kernel scaffold; baseline (speedup 1.0000x reference)
import jax
import jax.numpy as jnp
from jax.experimental import pallas as pl


def kernel(x, edge_index, edge_weight):
    raise NotImplementedError("write your pallas kernel here")


# speedup vs baseline: 4.1049x; 4.1049x over previous
import functools

import jax
import jax.numpy as jnp
from jax import lax
from jax.experimental import pallas as pl
from jax.experimental.pallas import tpu as pltpu
from jax.experimental.pallas import tpu_sc as plsc

N = 10000
D = 128
E = 320000
NC = 2
NS = 16
NW = NC * NS
C = 128
CPT = 79
EPT = CPT * C
E_PAD = NW * EPT
NPAD = 10240
RPT = NPAD // NS


def _spmm_body(x_hbm, src_hbm, dst_hbm, w_hbm, zero_hbm, acc_out,
               src_v, dst_v, w_v, rows, acc_sh, gsem, ssem):
    c = lax.axis_index("c")
    s = lax.axis_index("s")
    wid = s * NC + c

    pltpu.sync_copy(zero_hbm.at[pl.ds(s * RPT, RPT)],
                    acc_sh.at[pl.ds(s * RPT, RPT)])
    pltpu.sync_copy(src_hbm.at[wid], src_v)
    pltpu.sync_copy(dst_hbm.at[wid], dst_v)
    pltpu.sync_copy(w_hbm.at[wid], w_v)
    plsc.subcore_barrier()

    def chunk_body(k):
        pltpu.async_copy(x_hbm.at[src_v.at[k]], rows, gsem).wait()

        def group_body(g):
            w16 = w_v[k, pl.ds(g * 16, 16)]
            for i in range(16):
                ws = w16[i]
                r = g * 16 + i
                for j in range(D // 16):
                    sl = pl.ds(j * 16, 16)
                    rows[r, sl] = rows[r, sl] * ws
        pl.loop(0, C // 16)(group_body)

        pltpu.async_copy(rows, acc_sh.at[dst_v.at[k]], ssem, add=True).wait()

    pl.loop(0, CPT)(chunk_body)

    plsc.subcore_barrier()
    pltpu.sync_copy(acc_sh.at[pl.ds(s * RPT, RPT)],
                    acc_out.at[c, pl.ds(s * RPT, RPT)])


@jax.jit
def _spmm(x, srcp, dstp, wp, zeros):
    mesh = plsc.VectorSubcoreMesh(core_axis_name="c", subcore_axis_name="s")
    return pl.kernel(
        _spmm_body,
        out_type=jax.ShapeDtypeStruct((NC, NPAD, D), jnp.float32),
        mesh=mesh,
        scratch_types=[
            pltpu.VMEM((CPT, C), jnp.int32),
            pltpu.VMEM((CPT, C), jnp.int32),
            pltpu.VMEM((CPT, C), jnp.float32),
            pltpu.VMEM((C, D), jnp.float32),
            pltpu.VMEM_SHARED((NPAD, D), jnp.float32),
            pltpu.SemaphoreType.DMA,
            pltpu.SemaphoreType.DMA,
        ],
    )(x, srcp, dstp, wp, zeros)


def _combine_body(a_ref, b_ref, f_ref, h_out, f_out):
    h = a_ref[...] + b_ref[...]
    h_out[...] = h
    f_out[...] = f_ref[...] + h


@jax.jit
def _combine(acc, f_prev):
    blk = 1000
    grid = N // blk
    spec = pl.BlockSpec((blk, D), lambda i: (i, 0))
    return pl.pallas_call(
        _combine_body,
        grid=(grid,),
        in_specs=[spec, spec, spec],
        out_specs=[spec, spec],
        out_shape=[jax.ShapeDtypeStruct((N, D), jnp.float32),
                   jax.ShapeDtypeStruct((N, D), jnp.float32)],
    )(acc[0], acc[1], f_prev)


def kernel(x, edge_index, edge_weight):
    dst = edge_index[0]
    src = edge_index[1]
    pad = E_PAD - E
    srcp = jnp.concatenate([src, jnp.zeros((pad,), jnp.int32)]).reshape(NW, CPT, C)
    dstp = jnp.concatenate([dst, jnp.zeros((pad,), jnp.int32)]).reshape(NW, CPT, C)
    wp = jnp.concatenate([edge_weight, jnp.zeros((pad,), jnp.float32)]
                         ).reshape(NW, CPT, C)
    zeros = jnp.zeros((NPAD, D), jnp.float32)

    acc1 = _spmm(x, srcp, dstp, wp, zeros)
    h1, fin1 = _combine(acc1, x)
    acc2 = _spmm(h1, srcp, dstp, wp, zeros)
    _, fin = _combine(acc2, fin1)
    return fin


# speedup vs baseline: 5.2637x; 1.2823x over previous
import jax
import jax.numpy as jnp
from jax import lax
from jax.experimental import pallas as pl
from jax.experimental.pallas import tpu as pltpu
from jax.experimental.pallas import tpu_sc as plsc

N = 10000
D = 128
E = 320000
NC = 2
NS = 16
NW = NC * NS
C = 128
CPT = 79
EPT = CPT * C
E_PAD = NW * EPT
NPAD = 10240
RPT = NPAD // NS
NMETA = 4


def _spmm_body(x_hbm, meta_hbm, w_hbm, zero_hbm, acc_out,
               meta_v, w_v, rows, acc_sh,
               msem0, msem1, msem2, msem3, gsem0, gsem1, ssem0, ssem1):
    c = lax.axis_index("c")
    s = lax.axis_index("s")
    wid = s * NC + c

    pltpu.sync_copy(zero_hbm.at[pl.ds(s * RPT, RPT)],
                    acc_sh.at[pl.ds(s * RPT, RPT)])
    plsc.subcore_barrier()

    msems = (msem0, msem1, msem2, msem3)
    gsems = (gsem0, gsem1)
    ssems = (ssem0, ssem1)

    def issue_meta(k, m):
        pltpu.async_copy(meta_hbm.at[wid, k], meta_v.at[m], msems[m])
        pltpu.async_copy(w_hbm.at[wid, k], w_v.at[m], msems[m])

    def wait_meta(m):
        pltpu.make_async_copy(meta_hbm.at[wid, 0], meta_v.at[m],
                              msems[m]).wait()
        pltpu.make_async_copy(w_hbm.at[wid, 0], w_v.at[m],
                              msems[m]).wait()

    def issue_gather(m, b):
        pltpu.async_copy(x_hbm.at[meta_v.at[m, 0]], rows.at[b], gsems[b])

    def wait_gather(b):
        pltpu.make_async_copy(x_hbm.at[meta_v.at[0, 0]], rows.at[b],
                              gsems[b]).wait()

    def issue_scatter(m, b):
        pltpu.async_copy(rows.at[b], acc_sh.at[meta_v.at[m, 1]], ssems[b],
                         add=True)

    def wait_scatter(b):
        pltpu.make_async_copy(rows.at[b], acc_sh.at[meta_v.at[0, 1]],
                              ssems[b]).wait()

    def scale(m, b):
        def group_body(g):
            w16 = w_v[m, pl.ds(g * 16, 16)]
            for i in range(16):
                ws = w16[i]
                r = g * 16 + i
                for j in range(D // 16):
                    sl = pl.ds(j * 16, 16)
                    rows[b, r, sl] = rows[b, r, sl] * ws
        pl.loop(0, C // 16)(group_body)

    issue_meta(0, 0)
    issue_meta(1, 1)
    wait_meta(0)
    issue_gather(0, 0)

    def t_body(t):
        for q in range(NMETA):
            k = NMETA * t + q
            b = q % 2
            nb = 1 - b
            m = q
            m1 = (q + 1) % NMETA
            m2 = (q + 2) % NMETA

            @pl.when(k < CPT)
            def _(k=k, b=b, nb=nb, m=m, m1=m1, m2=m2):
                @pl.when(k + 2 < CPT)
                def _():
                    issue_meta(k + 2, m2)

                @pl.when(k + 1 < CPT)
                def _():
                    wait_meta(m1)

                    @pl.when(k >= 1)
                    def _():
                        wait_scatter(nb)
                    issue_gather(m1, nb)
                wait_gather(b)
                scale(m, b)
                issue_scatter(m, b)

    pl.loop(0, (CPT + NMETA - 1) // NMETA)(t_body)
    wait_scatter(0)
    wait_scatter(1)

    plsc.subcore_barrier()
    pltpu.sync_copy(acc_sh.at[pl.ds(s * RPT, RPT)],
                    acc_out.at[c, pl.ds(s * RPT, RPT)])


@jax.jit
def _spmm(x, meta, w, zeros):
    mesh = plsc.VectorSubcoreMesh(core_axis_name="c", subcore_axis_name="s")
    return pl.kernel(
        _spmm_body,
        out_type=jax.ShapeDtypeStruct((NC, NPAD, D), jnp.float32),
        mesh=mesh,
        scratch_types=[
            pltpu.VMEM((NMETA, 2, C), jnp.int32),
            pltpu.VMEM((NMETA, C), jnp.float32),
            pltpu.VMEM((2, C, D), jnp.float32),
            pltpu.VMEM_SHARED((NPAD, D), jnp.float32),
            pltpu.SemaphoreType.DMA,
            pltpu.SemaphoreType.DMA,
            pltpu.SemaphoreType.DMA,
            pltpu.SemaphoreType.DMA,
            pltpu.SemaphoreType.DMA,
            pltpu.SemaphoreType.DMA,
            pltpu.SemaphoreType.DMA,
            pltpu.SemaphoreType.DMA,
        ],
    )(x, meta, w, zeros)


def _combine_body(a_ref, b_ref, f_ref, h_out, f_out):
    h = a_ref[...] + b_ref[...]
    h_out[...] = h
    f_out[...] = f_ref[...] + h


@jax.jit
def _combine(acc, f_prev):
    blk = 1000
    grid = N // blk
    spec = pl.BlockSpec((blk, D), lambda i: (i, 0))
    return pl.pallas_call(
        _combine_body,
        grid=(grid,),
        in_specs=[spec, spec, spec],
        out_specs=[spec, spec],
        out_shape=[jax.ShapeDtypeStruct((N, D), jnp.float32),
                   jax.ShapeDtypeStruct((N, D), jnp.float32)],
    )(acc[0], acc[1], f_prev)


def kernel(x, edge_index, edge_weight):
    dst = edge_index[0]
    src = edge_index[1]
    pad = E_PAD - E
    srcp = jnp.concatenate([src, jnp.zeros((pad,), jnp.int32)]).reshape(NW, CPT, C)
    dstp = jnp.concatenate([dst, jnp.zeros((pad,), jnp.int32)]).reshape(NW, CPT, C)
    wp = jnp.concatenate([edge_weight, jnp.zeros((pad,), jnp.float32)]
                         ).reshape(NW, CPT, C)
    meta = jnp.stack([srcp, dstp], axis=2)
    zeros = jnp.zeros((NPAD, D), jnp.float32)

    acc1 = _spmm(x, meta, wp, zeros)
    h1, fin1 = _combine(acc1, x)
    acc2 = _spmm(h1, meta, wp, zeros)
    _, fin = _combine(acc2, fin1)
    return fin


# speedup vs baseline: 7.7392x; 1.4703x over previous
import jax
import jax.numpy as jnp
from jax import lax
from jax.experimental import pallas as pl
from jax.experimental.pallas import tpu as pltpu
from jax.experimental.pallas import tpu_sc as plsc

N = 10000
D = 128
E = 320000
NC = 2
NS = 16
NW = NC * NS
C = 128
CPT0 = 108
CPT1 = 49
CPTMAX = max(CPT0, CPT1)
E_PAD = 16 * (CPT0 + CPT1) * C
NPAD = 10240
RPT = NPAD // NS
NMETA = 4


def _spmm_body(x_hbm, meta_hbm, w_hbm, zero_hbm, acc_out,
               meta_v, w_v, rows, acc_sh,
               msem0, msem1, msem2, msem3, gsem0, gsem1, ssem0, ssem1):
    c = lax.axis_index("c")
    s = lax.axis_index("s")
    wid = s * NC + c

    ncpt = jnp.where(c == 0, CPT0, CPT1)

    pltpu.sync_copy(zero_hbm.at[pl.ds(s * RPT, RPT)],
                    acc_sh.at[pl.ds(s * RPT, RPT)])
    plsc.subcore_barrier()

    msems = (msem0, msem1, msem2, msem3)
    gsems = (gsem0, gsem1)
    ssems = (ssem0, ssem1)

    def issue_meta(k, m):
        pltpu.async_copy(meta_hbm.at[wid, k], meta_v.at[m], msems[m])
        pltpu.async_copy(w_hbm.at[wid, k], w_v.at[m], msems[m])

    def wait_meta(m):
        pltpu.make_async_copy(meta_hbm.at[wid, 0], meta_v.at[m],
                              msems[m]).wait()
        pltpu.make_async_copy(w_hbm.at[wid, 0], w_v.at[m],
                              msems[m]).wait()

    def issue_gather(m, b):
        pltpu.async_copy(x_hbm.at[meta_v.at[m, 0]], rows.at[b], gsems[b])

    def wait_gather(b):
        pltpu.make_async_copy(x_hbm.at[meta_v.at[0, 0]], rows.at[b],
                              gsems[b]).wait()

    def issue_scatter(m, b):
        pltpu.async_copy(rows.at[b], acc_sh.at[meta_v.at[m, 1]], ssems[b],
                         add=True)

    def wait_scatter(b):
        pltpu.make_async_copy(rows.at[b], acc_sh.at[meta_v.at[0, 1]],
                              ssems[b]).wait()

    def scale(m, b):
        def group_body(g):
            w16 = w_v[m, pl.ds(g * 16, 16)]
            for i in range(16):
                ws = w16[i]
                r = g * 16 + i
                for j in range(D // 16):
                    sl = pl.ds(j * 16, 16)
                    rows[b, r, sl] = rows[b, r, sl] * ws
        pl.loop(0, C // 16)(group_body)

    issue_meta(0, 0)
    issue_meta(1, 1)
    wait_meta(0)
    issue_gather(0, 0)

    def t_body(t):
        for q in range(NMETA):
            k = NMETA * t + q
            b = q % 2
            nb = 1 - b
            m = q
            m1 = (q + 1) % NMETA
            m2 = (q + 2) % NMETA

            @pl.when(k < ncpt)
            def _(k=k, b=b, nb=nb, m=m, m1=m1, m2=m2):
                @pl.when(k + 2 < ncpt)
                def _():
                    issue_meta(k + 2, m2)

                @pl.when(k + 1 < ncpt)
                def _():
                    wait_meta(m1)

                    @pl.when(k >= 1)
                    def _():
                        wait_scatter(nb)
                    issue_gather(m1, nb)
                wait_gather(b)
                scale(m, b)
                issue_scatter(m, b)

    pl.loop(0, (ncpt + NMETA - 1) // NMETA)(t_body)
    wait_scatter(0)
    wait_scatter(1)

    plsc.subcore_barrier()
    pltpu.sync_copy(acc_sh.at[pl.ds(s * RPT, RPT)],
                    acc_out.at[c, pl.ds(s * RPT, RPT)])


@jax.jit
def _spmm(x, meta, w, zeros):
    mesh = plsc.VectorSubcoreMesh(core_axis_name="c", subcore_axis_name="s")
    return pl.kernel(
        _spmm_body,
        out_type=jax.ShapeDtypeStruct((NC, NPAD, D), jnp.float32),
        mesh=mesh,
        scratch_types=[
            pltpu.VMEM((NMETA, 2, C), jnp.int32),
            pltpu.VMEM((NMETA, C), jnp.float32),
            pltpu.VMEM((2, C, D), jnp.float32),
            pltpu.VMEM_SHARED((NPAD, D), jnp.float32),
            pltpu.SemaphoreType.DMA,
            pltpu.SemaphoreType.DMA,
            pltpu.SemaphoreType.DMA,
            pltpu.SemaphoreType.DMA,
            pltpu.SemaphoreType.DMA,
            pltpu.SemaphoreType.DMA,
            pltpu.SemaphoreType.DMA,
            pltpu.SemaphoreType.DMA,
        ],
    )(x, meta, w, zeros)


def _combine_body(a_ref, b_ref, f_ref, h_out, f_out):
    h = a_ref[...] + b_ref[...]
    h_out[...] = h
    f_out[...] = f_ref[...] + h


@jax.jit
def _combine(acc, f_prev):
    blk = 1000
    grid = N // blk
    spec = pl.BlockSpec((blk, D), lambda i: (i, 0))
    return pl.pallas_call(
        _combine_body,
        grid=(grid,),
        in_specs=[spec, spec, spec],
        out_specs=[spec, spec],
        out_shape=[jax.ShapeDtypeStruct((N, D), jnp.float32),
                   jax.ShapeDtypeStruct((N, D), jnp.float32)],
    )(acc[0], acc[1], f_prev)


def _layout(arr, pad_val):
    pad = E_PAD - E
    flat = jnp.concatenate([arr, jnp.full((pad,), pad_val, arr.dtype)])
    ch = flat.reshape(-1, C)
    c0 = ch[:16 * CPT0].reshape(16, CPT0, C)
    c0 = jnp.pad(c0, ((0, 0), (0, CPTMAX - CPT0), (0, 0)))
    c1 = ch[16 * CPT0:].reshape(16, CPT1, C)
    c1 = jnp.pad(c1, ((0, 0), (0, CPTMAX - CPT1), (0, 0)))
    return jnp.stack([c0, c1], axis=1).reshape(NW, CPTMAX, C)


def kernel(x, edge_index, edge_weight):
    dst = edge_index[0]
    src = edge_index[1]
    srcp = _layout(src, 0)
    dstp = _layout(dst, 0)
    wp = _layout(edge_weight, 0.0)
    meta = jnp.stack([srcp, dstp], axis=2)
    zeros = jnp.zeros((NPAD, D), jnp.float32)

    acc1 = _spmm(x, meta, wp, zeros)
    h1, fin1 = _combine(acc1, x)
    acc2 = _spmm(h1, meta, wp, zeros)
    _, fin = _combine(acc2, fin1)
    return fin


# speedup vs baseline: 7.9276x; 1.0243x over previous
import jax
import jax.numpy as jnp
from jax import lax
from jax.experimental import pallas as pl
from jax.experimental.pallas import tpu as pltpu
from jax.experimental.pallas import tpu_sc as plsc

N = 10000
D = 128
E = 320000
NC = 2
NS = 16
NW = NC * NS
C = 96
CPT0 = 144
CPT1 = 65
CPTMAX = max(CPT0, CPT1)
E_PAD = 16 * (CPT0 + CPT1) * C
NBUF = 3
NPAD = 10240
RPT = NPAD // NS
NMETA = 6


def _spmm_body(x_hbm, meta_hbm, w_hbm, zero_hbm, acc_out,
               meta_v, w_v, rows, acc_sh,
               msem0, msem1, msem2, msem3, msem4, msem5,
               gsem0, gsem1, gsem2, ssem0, ssem1, ssem2):
    c = lax.axis_index("c")
    s = lax.axis_index("s")
    wid = s * NC + c

    ncpt = jnp.where(c == 0, CPT0, CPT1)


    msems = (msem0, msem1, msem2, msem3, msem4, msem5)
    gsems = (gsem0, gsem1, gsem2)
    ssems = (ssem0, ssem1, ssem2)

    def issue_meta(k, m):
        pltpu.async_copy(meta_hbm.at[wid, k], meta_v.at[m], msems[m])
        pltpu.async_copy(w_hbm.at[wid, k], w_v.at[m], msems[m])

    def wait_meta(m):
        pltpu.make_async_copy(meta_hbm.at[wid, 0], meta_v.at[m],
                              msems[m]).wait()
        pltpu.make_async_copy(w_hbm.at[wid, 0], w_v.at[m],
                              msems[m]).wait()

    def issue_gather(m, b):
        pltpu.async_copy(x_hbm.at[meta_v.at[m, 0]], rows.at[b], gsems[b])

    def wait_gather(b):
        pltpu.make_async_copy(x_hbm.at[meta_v.at[0, 0]], rows.at[b],
                              gsems[b]).wait()

    def issue_scatter(m, b):
        pltpu.async_copy(rows.at[b], acc_sh.at[meta_v.at[m, 1]], ssems[b],
                         add=True)

    def wait_scatter(b):
        pltpu.make_async_copy(rows.at[b], acc_sh.at[meta_v.at[0, 1]],
                              ssems[b]).wait()

    def scale(m, b):
        def group_body(g):
            w16 = w_v[m, pl.ds(g * 16, 16)]
            for i in range(16):
                ws = w16[i]
                r = g * 16 + i
                for j in range(D // 16):
                    sl = pl.ds(j * 16, 16)
                    rows[b, r, sl] = rows[b, r, sl] * ws
        pl.loop(0, C // 16)(group_body)

    issue_meta(0, 0)
    issue_meta(1, 1)
    pltpu.sync_copy(zero_hbm.at[pl.ds(s * RPT, RPT)],
                    acc_sh.at[pl.ds(s * RPT, RPT)])
    plsc.subcore_barrier()
    wait_meta(0)
    issue_gather(0, 0)

    def t_body(t):
        for q in range(NMETA):
            k = NMETA * t + q
            b = q % NBUF
            m = q
            m1 = (q + 1) % NMETA
            m2 = (q + 2) % NMETA
            b1 = (q + 1) % NBUF

            @pl.when(k < ncpt)
            def _(k=k, b=b, m=m, m1=m1, m2=m2, b1=b1):
                @pl.when(k + 2 < ncpt)
                def _():
                    issue_meta(k + 2, m2)

                @pl.when(k + 1 < ncpt)
                def _():
                    wait_meta(m1)

                    @pl.when(k >= 2)
                    def _():
                        wait_scatter(b1)
                    issue_gather(m1, b1)
                wait_gather(b)
                scale(m, b)
                issue_scatter(m, b)

    pl.loop(0, (ncpt + NMETA - 1) // NMETA)(t_body)
    wait_scatter(0)
    wait_scatter(1)
    wait_scatter(2)

    plsc.subcore_barrier()
    pltpu.sync_copy(acc_sh.at[pl.ds(s * RPT, RPT)],
                    acc_out.at[c, pl.ds(s * RPT, RPT)])


@jax.jit
def _spmm(x, meta, w, zeros):
    mesh = plsc.VectorSubcoreMesh(core_axis_name="c", subcore_axis_name="s")
    return pl.kernel(
        _spmm_body,
        out_type=jax.ShapeDtypeStruct((NC, NPAD, D), jnp.float32),
        mesh=mesh,
        scratch_types=[
            pltpu.VMEM((NMETA, 2, C), jnp.int32),
            pltpu.VMEM((NMETA, C), jnp.float32),
            pltpu.VMEM((NBUF, C, D), jnp.float32),
            pltpu.VMEM_SHARED((NPAD, D), jnp.float32),
        ] + [pltpu.SemaphoreType.DMA] * (NMETA + 2 * NBUF),
    )(x, meta, w, zeros)


def _combine_body(a_ref, b_ref, f_ref, h_out, f_out):
    h = a_ref[...] + b_ref[...]
    h_out[...] = h
    f_out[...] = f_ref[...] + h


@jax.jit
def _combine(acc, f_prev):
    blk = 1000
    grid = N // blk
    spec = pl.BlockSpec((blk, D), lambda i: (i, 0))
    return pl.pallas_call(
        _combine_body,
        grid=(grid,),
        in_specs=[spec, spec, spec],
        out_specs=[spec, spec],
        out_shape=[jax.ShapeDtypeStruct((N, D), jnp.float32),
                   jax.ShapeDtypeStruct((N, D), jnp.float32)],
    )(acc[0], acc[1], f_prev)


def _layout(arr, pad_val):
    pad = E_PAD - E
    flat = jnp.concatenate([arr, jnp.full((pad,), pad_val, arr.dtype)])
    ch = flat.reshape(-1, C)
    c0 = ch[:16 * CPT0].reshape(16, CPT0, C)
    c0 = jnp.pad(c0, ((0, 0), (0, CPTMAX - CPT0), (0, 0)))
    c1 = ch[16 * CPT0:].reshape(16, CPT1, C)
    c1 = jnp.pad(c1, ((0, 0), (0, CPTMAX - CPT1), (0, 0)))
    return jnp.stack([c0, c1], axis=1).reshape(NW, CPTMAX, C)


def kernel(x, edge_index, edge_weight):
    dst = edge_index[0]
    src = edge_index[1]
    srcp = _layout(src, 0)
    dstp = _layout(dst, 0)
    wp = _layout(edge_weight, 0.0)
    meta = jnp.stack([srcp, dstp], axis=2)
    zeros = jnp.zeros((NPAD, D), jnp.float32)

    acc1 = _spmm(x, meta, wp, zeros)
    h1, fin1 = _combine(acc1, x)
    acc2 = _spmm(h1, meta, wp, zeros)
    _, fin = _combine(acc2, fin1)
    return fin


# speedup vs baseline: 9.4890x; 1.1970x over previous
import jax
import jax.numpy as jnp
from jax import lax
from jax.experimental import pallas as pl
from jax.experimental.pallas import tpu as pltpu
from jax.experimental.pallas import tpu_sc as plsc

N = 10000
D = 128
E = 320000
NC = 2
NS = 16
NW = NC * NS
C = 96
NCHUNK = (E + C - 1) // C
E_EXT = NCHUNK * C
CPT0 = 132
CPT1 = 76
XTRA = NCHUNK - 16 * (CPT0 + CPT1)
NPAD = 10240
RPT = NPAD // NS
NBUF = 3
NMETA = 6


def _spmm_body(x_hbm, src_hbm, dst_hbm, w_hbm, zero_hbm, acc_out,
               sv, dv, wv, rows, acc_sh,
               msem0, msem1, msem2, msem3, msem4, msem5,
               gsem0, gsem1, gsem2, ssem0, ssem1, ssem2):
    c = lax.axis_index("c")
    s = lax.axis_index("s")

    ncpt = jnp.where(c == 0, CPT0, CPT1 + jnp.where(s < XTRA, 1, 0))
    toff = jnp.where(c == 0, s * CPT0,
                     16 * CPT0 + s * CPT1 + jnp.minimum(s, XTRA))

    msems = (msem0, msem1, msem2, msem3, msem4, msem5)
    gsems = (gsem0, gsem1, gsem2)
    ssems = (ssem0, ssem1, ssem2)

    def issue_meta(k, m):
        off = pl.multiple_of((toff + k) * C, 8)
        pltpu.async_copy(src_hbm.at[pl.ds(off, C)], sv.at[m], msems[m])
        pltpu.async_copy(dst_hbm.at[pl.ds(off, C)], dv.at[m], msems[m])
        pltpu.async_copy(w_hbm.at[pl.ds(off, C)], wv.at[m], msems[m])

    def wait_meta(m):
        pltpu.make_async_copy(src_hbm.at[pl.ds(0, C)], sv.at[m],
                              msems[m]).wait()
        pltpu.make_async_copy(dst_hbm.at[pl.ds(0, C)], dv.at[m],
                              msems[m]).wait()
        pltpu.make_async_copy(w_hbm.at[pl.ds(0, C)], wv.at[m],
                              msems[m]).wait()

    def issue_gather(m, b):
        pltpu.async_copy(x_hbm.at[sv.at[m]], rows.at[b], gsems[b])

    def wait_gather(b):
        pltpu.make_async_copy(x_hbm.at[sv.at[0]], rows.at[b],
                              gsems[b]).wait()

    def issue_scatter(m, b):
        pltpu.async_copy(rows.at[b], acc_sh.at[dv.at[m]], ssems[b],
                         add=True)

    def wait_scatter(b):
        pltpu.make_async_copy(rows.at[b], acc_sh.at[dv.at[0]],
                              ssems[b]).wait()

    def scale(m, b):
        def group_body(g):
            w16 = wv[m, pl.ds(g * 16, 16)]
            for i in range(16):
                ws = w16[i]
                r = g * 16 + i
                for j in range(D // 16):
                    sl = pl.ds(j * 16, 16)
                    rows[b, r, sl] = rows[b, r, sl] * ws
        pl.loop(0, C // 16)(group_body)

    issue_meta(0, 0)
    issue_meta(1, 1)
    pltpu.sync_copy(zero_hbm.at[pl.ds(s * RPT, RPT)],
                    acc_sh.at[pl.ds(s * RPT, RPT)])
    plsc.subcore_barrier()
    wait_meta(0)
    issue_gather(0, 0)

    def t_body(t):
        for q in range(NMETA):
            k = NMETA * t + q
            b = q % NBUF
            m = q
            m1 = (q + 1) % NMETA
            m2 = (q + 2) % NMETA
            b1 = (q + 1) % NBUF

            @pl.when(k < ncpt)
            def _(k=k, b=b, m=m, m1=m1, m2=m2, b1=b1):
                @pl.when(k + 2 < ncpt)
                def _():
                    issue_meta(k + 2, m2)

                @pl.when(k + 1 < ncpt)
                def _():
                    wait_meta(m1)

                    @pl.when(k >= 2)
                    def _():
                        wait_scatter(b1)
                    issue_gather(m1, b1)
                wait_gather(b)
                scale(m, b)
                issue_scatter(m, b)

    pl.loop(0, (ncpt + NMETA - 1) // NMETA)(t_body)
    wait_scatter(0)
    wait_scatter(1)
    wait_scatter(2)

    plsc.subcore_barrier()
    pltpu.sync_copy(acc_sh.at[pl.ds(s * RPT, RPT)],
                    acc_out.at[c, pl.ds(s * RPT, RPT)])


@jax.jit
def _spmm(x, src, dst, w, zeros):
    mesh = plsc.VectorSubcoreMesh(core_axis_name="c", subcore_axis_name="s")
    return pl.kernel(
        _spmm_body,
        out_type=jax.ShapeDtypeStruct((NC, NPAD, D), jnp.float32),
        mesh=mesh,
        scratch_types=[
            pltpu.VMEM((NMETA, C), jnp.int32),
            pltpu.VMEM((NMETA, C), jnp.int32),
            pltpu.VMEM((NMETA, C), jnp.float32),
            pltpu.VMEM((NBUF, C, D), jnp.float32),
            pltpu.VMEM_SHARED((NPAD, D), jnp.float32),
        ] + [pltpu.SemaphoreType.DMA] * (NMETA + 2 * NBUF),
    )(x, src, dst, w, zeros)


def _combine_body(a_ref, b_ref, f_ref, h_out, f_out):
    h = a_ref[0] + b_ref[0]
    h_out[...] = h
    f_out[...] = f_ref[...] + h


@jax.jit
def _combine(acc, f_prev):
    blk = 1000
    grid = N // blk
    spec2 = pl.BlockSpec((blk, D), lambda i: (i, 0))
    return pl.pallas_call(
        _combine_body,
        grid=(grid,),
        in_specs=[pl.BlockSpec((1, blk, D), lambda i: (0, i, 0)),
                  pl.BlockSpec((1, blk, D), lambda i: (1, i, 0)),
                  spec2],
        out_specs=[spec2, spec2],
        out_shape=[jax.ShapeDtypeStruct((N, D), jnp.float32),
                   jax.ShapeDtypeStruct((N, D), jnp.float32)],
    )(acc, acc, f_prev)


def kernel(x, edge_index, edge_weight):
    pad = E_EXT - E
    dst = jnp.concatenate([edge_index[0], jnp.zeros((pad,), jnp.int32)])
    src = jnp.concatenate([edge_index[1], jnp.zeros((pad,), jnp.int32)])
    w = jnp.concatenate([edge_weight, jnp.zeros((pad,), jnp.float32)])
    zeros = jnp.zeros((NPAD, D), jnp.float32)

    acc1 = _spmm(x, src, dst, w, zeros)
    h1, fin1 = _combine(acc1, x)
    acc2 = _spmm(h1, src, dst, w, zeros)
    _, fin = _combine(acc2, fin1)
    return fin
